# baseline (speedup 1.0000x reference)
import functools

import jax
import jax.numpy as jnp
from jax import lax
from jax.experimental import pallas as pl
from jax.experimental.pallas import tpu as pltpu
from jax.experimental.pallas import tpu_sc as plsc

_NUM_BINS = 256
_NC, _NS, _LANES = 2, 16, 16
_NW = _NC * _NS
_ROW_TILE = 8
_COL_TILE = 128
_MAX_CHUNK_COLS = 2688


def _pick_chunk_cols(cols_main: int) -> int:
    best = _COL_TILE
    for cols in range(_COL_TILE, _MAX_CHUNK_COLS + 1, _COL_TILE):
        if cols_main % cols == 0:
            best = cols
    return best


@functools.lru_cache(maxsize=None)
def _make_sc_hist(length: int, num_channels: int):
    assert num_channels % _ROW_TILE == 0
    n_rt = num_channels // _ROW_TILE
    assert _NW % n_rt == 0
    n_q = _NW // n_rt
    cols_main = (length // (n_q * _COL_TILE)) * _COL_TILE
    covered = n_q * cols_main
    tail = length - covered
    assert 0 <= tail < _COL_TILE and tail % _LANES == 0
    chunk_c = _pick_chunk_cols(cols_main)
    nchunks = cols_main // chunk_c
    npairs = nchunks // 2
    odd_tail_chunk = nchunks % 2 == 1

    bank = _ROW_TILE * _NUM_BINS * _LANES
    hist_words = 2 * bank
    out_words = _ROW_TILE * _NUM_BINS

    mesh = plsc.VectorSubcoreMesh(
        core_axis_name="c", subcore_axis_name="s",
        num_cores=_NC, num_subcores=_NS)

    @functools.partial(
        pl.kernel,
        out_type=jax.ShapeDtypeStruct((_NW * out_words,), jnp.int32),
        mesh=mesh,
        scratch_types=[
            pltpu.VMEM((_ROW_TILE, chunk_c), jnp.int32),
            pltpu.VMEM((_ROW_TILE, chunk_c), jnp.int32),
            pltpu.VMEM((hist_words,), jnp.int32),
            pltpu.VMEM((out_words,), jnp.int32),
            pltpu.SemaphoreType.DMA,
            pltpu.SemaphoreType.DMA,
        ],
        compiler_params=pltpu.CompilerParams(
            needs_layout_passes=False, use_tc_tiling_on_sc=True),
    )
    def hist_kernel(arr, *rest):
        if tail:
            tailarr, out, buf0, buf1, hist, outbuf, sem0, sem1 = rest
        else:
            out, buf0, buf1, hist, outbuf, sem0, sem1 = rest
        cid = lax.axis_index("c")
        sid = lax.axis_index("s")
        wid = sid * _NC + cid
        q = wid // n_rt
        rt = wid % n_rt
        row0 = rt * _ROW_TILE
        col_base = q * cols_main

        zeros16 = jnp.zeros((_LANES,), jnp.int32)

        @plsc.parallel_loop(0, hist_words // _LANES, unroll=8)
        def zero_body(i):
            hist[pl.ds(i * _LANES, _LANES)] = zeros16

        lane = lax.iota(jnp.int32, _LANES)
        cbs = [lane + c8 * (_NUM_BINS * _LANES) for c8 in range(_ROW_TILE)]
        ones16 = jnp.ones((_LANES,), jnp.int32)

        def process(buf, ncols, xvec):
            nduo = ncols // (2 * _LANES)

            @plsc.parallel_loop(0, nduo, unroll=4)
            def g_body(g):
                base = 2 * g * _LANES
                for c8, cb in enumerate(cbs):
                    v = buf[c8, pl.ds(base, _LANES)]
                    idx = v * _LANES + cb
                    plsc.addupdate_scatter(hist, [idx], xvec)
                for c8, cb in enumerate(cbs):
                    v = buf[c8, pl.ds(base + _LANES, _LANES)]
                    idx = v * _LANES + cb + bank
                    plsc.addupdate_scatter(hist, [idx], xvec)

        def issue(c, buf, sem):
            off = pl.multiple_of(col_base + c * chunk_c, _COL_TILE)
            return pltpu.async_copy(
                arr.at[pl.ds(row0, _ROW_TILE), pl.ds(off, chunk_c)], buf, sem)

        def wait(buf, sem):
            pltpu.make_async_copy(
                arr.at[pl.ds(0, _ROW_TILE), pl.ds(0, chunk_c)], buf, sem
            ).wait()

        issue(0, buf0, sem0)
        if nchunks > 1:
            issue(1, buf1, sem1)

        def pair_body(k, carry):
            wait(buf0, sem0)
            process(buf0, chunk_c, ones16)

            @pl.when(2 * k + 2 < nchunks)
            def _i0():
                issue(2 * k + 2, buf0, sem0)

            wait(buf1, sem1)
            process(buf1, chunk_c, ones16)

            @pl.when(2 * k + 3 < nchunks)
            def _i1():
                issue(2 * k + 3, buf1, sem1)

            return carry

        lax.fori_loop(0, npairs, pair_body, 0)

        if odd_tail_chunk:
            wait(buf0, sem0)
            process(buf0, chunk_c, ones16)

        if tail:
            assert (_COL_TILE - tail) % (2 * _LANES) == 0
            pad_fix = jnp.full((_LANES,),
                               -((_COL_TILE - tail) // (2 * _LANES)),
                               jnp.int32)

            @pl.when(q == n_q - 1)
            def _tail():
                pltpu.sync_copy(
                    tailarr.at[pl.ds(row0, _ROW_TILE), pl.ds(0, _COL_TILE)],
                    buf0.at[pl.ds(0, _ROW_TILE), pl.ds(0, _COL_TILE)])
                process(buf0, _COL_TILE, ones16)
                for cb in cbs:
                    plsc.addupdate_scatter(hist, [cb], pad_fix)
                    plsc.addupdate_scatter(hist, [cb + bank], pad_fix)

        gidx = [lane * _LANES + l for l in range(_LANES)]

        @plsc.parallel_loop(0, out_words // _LANES)
        def red_body(i):
            b0 = i * _LANES
            off = b0 * _LANES
            acc = plsc.load_gather(hist, [gidx[0] + off])
            for l in range(1, _LANES):
                acc = acc + plsc.load_gather(hist, [gidx[l] + off])
            for l in range(_LANES):
                acc = acc + plsc.load_gather(hist, [gidx[l] + off + bank])
            outbuf[pl.ds(b0, _LANES)] = acc

        pltpu.sync_copy(outbuf, out.at[pl.ds(wid * out_words, out_words)])

    return hist_kernel


def kernel(array):
    length, num_channels = array.shape
    n_rt = num_channels // _ROW_TILE
    n_q = _NW // n_rt
    covered = n_q * ((length // (n_q * _COL_TILE)) * _COL_TILE)
    tail = length - covered
    args = [array.T]
    if tail:
        tail_t = lax.slice(array, (covered, 0), (length, num_channels)).T
        args.append(jnp.pad(tail_t, ((0, 0), (0, _COL_TILE - tail))))
    partials = _make_sc_hist(length, num_channels)(*args)
    return partials.reshape(n_q, num_channels, _NUM_BINS).sum(axis=0)


# speedup vs baseline: 1.0955x; 1.0955x over previous
import functools

import jax
import jax.numpy as jnp
from jax import lax
from jax.experimental import pallas as pl
from jax.experimental.pallas import tpu as pltpu
from jax.experimental.pallas import tpu_sc as plsc

_NUM_BINS = 256
_NC, _NS, _LANES = 2, 16, 16
_NW = _NC * _NS
_ROW_TILE = 8
_COL_TILE = 128
_MAX_CHUNK_COLS = 6016


def _pick_chunk_cols(cols_main: int) -> int:
    best = _COL_TILE
    for cols in range(_COL_TILE, _MAX_CHUNK_COLS + 1, _COL_TILE):
        if cols_main % cols == 0:
            best = cols
    return best


@functools.lru_cache(maxsize=None)
def _make_sc_hist(length: int, num_channels: int):
    assert num_channels % _ROW_TILE == 0
    n_rt = num_channels // _ROW_TILE
    assert _NW % n_rt == 0
    n_q = _NW // n_rt
    cols_main = (length // (n_q * _COL_TILE)) * _COL_TILE
    covered = n_q * cols_main
    tail = length - covered
    assert 0 <= tail < _COL_TILE and tail % _LANES == 0
    chunk_c = _pick_chunk_cols(cols_main)
    nchunks = cols_main // chunk_c
    npairs = nchunks // 2
    odd_tail_chunk = nchunks % 2 == 1

    hist_words = _ROW_TILE * _NUM_BINS * _LANES
    out_words = _ROW_TILE * _NUM_BINS

    mesh = plsc.VectorSubcoreMesh(
        core_axis_name="c", subcore_axis_name="s",
        num_cores=_NC, num_subcores=_NS)

    @functools.partial(
        pl.kernel,
        out_type=jax.ShapeDtypeStruct((_NW * out_words,), jnp.int32),
        mesh=mesh,
        scratch_types=[
            pltpu.VMEM((_ROW_TILE, chunk_c), jnp.int32),
            pltpu.VMEM((_ROW_TILE, chunk_c), jnp.int32),
            pltpu.VMEM((hist_words,), jnp.int32),
            pltpu.VMEM((out_words,), jnp.int32),
            pltpu.SemaphoreType.DMA,
            pltpu.SemaphoreType.DMA,
        ],
        compiler_params=pltpu.CompilerParams(
            needs_layout_passes=False, use_tc_tiling_on_sc=True),
    )
    def hist_kernel(arr, *rest):
        if tail:
            tailarr, out, buf0, buf1, hist, outbuf, sem0, sem1 = rest
        else:
            out, buf0, buf1, hist, outbuf, sem0, sem1 = rest
        cid = lax.axis_index("c")
        sid = lax.axis_index("s")
        wid = sid * _NC + cid
        q = wid // n_rt
        rt = wid % n_rt
        row0 = rt * _ROW_TILE
        col_base = q * cols_main

        zeros16 = jnp.zeros((_LANES,), jnp.int32)

        @plsc.parallel_loop(0, hist_words // _LANES, unroll=8)
        def zero_body(i):
            hist[pl.ds(i * _LANES, _LANES)] = zeros16

        lane = lax.iota(jnp.int32, _LANES)
        cbs = [lane + c8 * (_NUM_BINS * _LANES) for c8 in range(_ROW_TILE)]
        ones16 = jnp.ones((_LANES,), jnp.int32)

        def process(buf, ncols, xvec):
            @plsc.parallel_loop(0, ncols // _LANES, unroll=8)
            def g_body(g):
                base = g * _LANES
                for c8, cb in enumerate(cbs):
                    v = buf[c8, pl.ds(base, _LANES)]
                    idx = v * _LANES + cb
                    plsc.addupdate_scatter(hist, [idx], xvec)

        def issue(c, buf, sem):
            off = pl.multiple_of(col_base + c * chunk_c, _COL_TILE)
            return pltpu.async_copy(
                arr.at[pl.ds(row0, _ROW_TILE), pl.ds(off, chunk_c)], buf, sem)

        def wait(buf, sem):
            pltpu.make_async_copy(
                arr.at[pl.ds(0, _ROW_TILE), pl.ds(0, chunk_c)], buf, sem
            ).wait()

        issue(0, buf0, sem0)
        if nchunks > 1:
            issue(1, buf1, sem1)

        def pair_body(k, carry):
            wait(buf0, sem0)
            process(buf0, chunk_c, ones16)

            @pl.when(2 * k + 2 < nchunks)
            def _i0():
                issue(2 * k + 2, buf0, sem0)

            wait(buf1, sem1)
            process(buf1, chunk_c, ones16)

            @pl.when(2 * k + 3 < nchunks)
            def _i1():
                issue(2 * k + 3, buf1, sem1)

            return carry

        lax.fori_loop(0, npairs, pair_body, 0)

        if odd_tail_chunk:
            wait(buf0, sem0)
            process(buf0, chunk_c, ones16)

        if tail:
            pad_fix = jnp.full((_LANES,), -((_COL_TILE - tail) // _LANES),
                               jnp.int32)

            @pl.when(q == n_q - 1)
            def _tail():
                pltpu.sync_copy(
                    tailarr.at[pl.ds(row0, _ROW_TILE), pl.ds(0, _COL_TILE)],
                    buf0.at[pl.ds(0, _ROW_TILE), pl.ds(0, _COL_TILE)])
                process(buf0, _COL_TILE, ones16)
                for cb in cbs:
                    plsc.addupdate_scatter(hist, [cb], pad_fix)

        gidx = [lane * _LANES + l for l in range(_LANES)]

        @plsc.parallel_loop(0, out_words // _LANES)
        def red_body(i):
            b0 = i * _LANES
            off = b0 * _LANES
            acc = plsc.load_gather(hist, [gidx[0] + off])
            for l in range(1, _LANES):
                acc = acc + plsc.load_gather(hist, [gidx[l] + off])
            outbuf[pl.ds(b0, _LANES)] = acc

        pltpu.sync_copy(outbuf, out.at[pl.ds(wid * out_words, out_words)])

    return hist_kernel


def kernel(array):
    length, num_channels = array.shape
    n_rt = num_channels // _ROW_TILE
    n_q = _NW // n_rt
    covered = n_q * ((length // (n_q * _COL_TILE)) * _COL_TILE)
    tail = length - covered
    args = [array.T]
    if tail:
        tail_t = lax.slice(array, (covered, 0), (length, num_channels)).T
        args.append(jnp.pad(tail_t, ((0, 0), (0, _COL_TILE - tail))))
    partials = _make_sc_hist(length, num_channels)(*args)
    return partials.reshape(n_q, num_channels, _NUM_BINS).sum(axis=0)
